# Initial kernel scaffold; baseline (speedup 1.0000x reference)
import jax
import jax.numpy as jnp
from jax.experimental import pallas as pl


def kernel(xyz1, xyz2):
    raise NotImplementedError("write your pallas kernel here")


# speedup vs baseline: 1.6684x; 1.6684x over previous
import jax
import jax.numpy as jnp
from jax import lax
from jax.experimental import pallas as pl

_BM = 512


def _chamfer_body(x1b_ref, x2b_ref, sq1_ref, xyz1_ref, xyz2_ref, out_ref):
    f32 = jnp.float32
    x1b = x1b_ref[0]
    sq1r = sq1_ref[0]
    xyz1 = xyz1_ref[0]
    n = x1b.shape[0]
    m = x2b_ref.shape[1]
    nt = m // _BM

    def tile1(t, carry):
        qbest, qarg, ccounts, d2sum = carry
        m0 = t * _BM
        x2blk = x2b_ref[0, pl.ds(m0, _BM), :]
        xyz2b = xyz2_ref[0, pl.ds(m0, _BM), :]
        inner = lax.dot_general(x2blk, x1b, (((1,), (1,)), ((), ())),
                                preferred_element_type=f32)
        sq2c = jnp.sum(xyz2b * xyz2b, axis=1, keepdims=True)
        dt = (sq2c + sq1r) - 2.0 * inner
        qmin = jnp.min(dt, axis=0, keepdims=True)
        riota = lax.broadcasted_iota(jnp.int32, (_BM, n), 0)
        qloc = jnp.min(jnp.where(dt == qmin, riota, _BM), axis=0,
                       keepdims=True)
        cond = qmin < qbest
        qbest = jnp.where(cond, qmin, qbest)
        qarg = jnp.where(cond, qloc + m0, qarg)
        cmin = jnp.min(dt, axis=1, keepdims=True)
        ciota = lax.broadcasted_iota(jnp.int32, (_BM, n), 1)
        nloc = jnp.min(jnp.where(dt == cmin, ciota, n), axis=1,
                       keepdims=True)
        onehot = (ciota == nloc).astype(f32)
        ccounts = ccounts + jnp.sum(onehot, axis=0, keepdims=True)
        d2sum = d2sum + jnp.sum(cmin)
        return qbest, qarg, ccounts, d2sum

    init = (jnp.full((1, n), jnp.inf, f32), jnp.zeros((1, n), jnp.int32),
            jnp.zeros((1, n), f32), jnp.float32(0.0))
    qbest, qarg, ccounts, d2sum = lax.fori_loop(0, nt, tile1, init)

    d1sum = jnp.sum(qbest)
    c1sum = lax.dot_general(ccounts, xyz1, (((1,), (0,)), ((), ())),
                            preferred_element_type=f32,
                            precision=lax.Precision.HIGHEST)

    def tile2(t, c2sum):
        m0 = t * _BM
        mids = lax.broadcasted_iota(jnp.int32, (_BM, 1), 0) + m0
        counts = jnp.sum((mids == qarg).astype(f32), axis=1,
                         keepdims=True)
        xyz2b = xyz2_ref[0, pl.ds(m0, _BM), :]
        return c2sum + lax.dot_general(counts, xyz2b, (((0,), (0,)), ((), ())),
                                       preferred_element_type=f32,
                                       precision=lax.Precision.HIGHEST)

    c2sum = lax.fori_loop(0, nt, tile2, jnp.zeros((1, 3), f32))

    s1 = jnp.sum(xyz1, axis=0, keepdims=True)
    s2 = jnp.sum(xyz2_ref[0], axis=0, keepdims=True)

    out = jnp.concatenate([
        d1sum.reshape(1, 1), d2sum.reshape(1, 1),
        c1sum, c2sum, s1, s2, jnp.zeros((1, 2), f32),
    ], axis=1)
    out_ref[...] = out.reshape(1, 1, 16)


@jax.jit
def kernel(xyz1, xyz2):
    b, n, _ = xyz1.shape
    m = xyz2.shape[1]
    f32 = jnp.float32
    xyz1 = xyz1.astype(f32)
    xyz2 = xyz2.astype(f32)
    pad1 = jnp.zeros((b, n, 5), f32)
    pad2 = jnp.zeros((b, m, 5), f32)
    x1b = jnp.concatenate([xyz1, pad1], axis=-1).astype(jnp.bfloat16)
    x2b = jnp.concatenate([xyz2, pad2], axis=-1).astype(jnp.bfloat16)
    sq1r = jnp.sum(xyz1 * xyz1, axis=-1)[:, None, :]

    partial = pl.pallas_call(
        _chamfer_body,
        grid=(b,),
        in_specs=[
            pl.BlockSpec((1, n, 8), lambda i: (i, 0, 0)),
            pl.BlockSpec((1, m, 8), lambda i: (i, 0, 0)),
            pl.BlockSpec((1, 1, n), lambda i: (i, 0, 0)),
            pl.BlockSpec((1, n, 3), lambda i: (i, 0, 0)),
            pl.BlockSpec((1, m, 3), lambda i: (i, 0, 0)),
        ],
        out_specs=pl.BlockSpec((1, 1, 16), lambda i: (i, 0, 0)),
        out_shape=jax.ShapeDtypeStruct((b, 1, 16), f32),
    )(x1b, x2b, sq1r, xyz1, xyz2)

    tot = jnp.sum(partial[:, 0, :], axis=0)
    d1sum, d2sum = tot[0], tot[1]
    c1sum = tot[2:5]
    c2sum = tot[5:8]
    s1 = tot[8:11]
    s2 = tot[11:14]
    cd = d1sum / (b * n) + d2sum / (b * m)
    dev = (s1 - c2sum) / (b * n) + (s2 - c1sum) / (b * m)
    return jnp.concatenate([cd[None], dev], axis=0)


# speedup vs baseline: 1.7471x; 1.0471x over previous
import functools

import jax
import jax.numpy as jnp
from jax import lax
from jax.experimental import pallas as pl
from jax.experimental.pallas import tpu as pltpu
from jax.experimental.pallas import tpu_sc as plsc

_BM = 512

_NC = 2
_NS = 16
_NW = _NC * _NS


def _chamfer_body(x1b_ref, x2b_ref, sq1_ref, xyz1_ref, xyz2_ref,
                  out_ref, idx1_ref, idx2_ref):
    f32 = jnp.float32
    x1b = x1b_ref[0]
    sq1r = sq1_ref[0]
    xyz1 = xyz1_ref[0]
    n = x1b.shape[0]
    m = x2b_ref.shape[1]
    nt = m // _BM

    def tile1(t, carry):
        qbest, qarg, d2sum = carry
        m0 = t * _BM
        x2blk = x2b_ref[0, pl.ds(m0, _BM), :]
        xyz2b = xyz2_ref[0, pl.ds(m0, _BM), :]
        inner = lax.dot_general(x2blk, x1b, (((1,), (1,)), ((), ())),
                                preferred_element_type=f32)
        sq2c = jnp.sum(xyz2b * xyz2b, axis=1, keepdims=True)
        dt = (sq2c + sq1r) - 2.0 * inner
        qmin = jnp.min(dt, axis=0, keepdims=True)
        riota = lax.broadcasted_iota(jnp.int32, (_BM, n), 0)
        qloc = jnp.min(jnp.where(dt == qmin, riota, _BM), axis=0,
                       keepdims=True)
        cond = qmin < qbest
        qbest = jnp.where(cond, qmin, qbest)
        qarg = jnp.where(cond, qloc + m0, qarg)
        cmin = jnp.min(dt, axis=1, keepdims=True)
        ciota = lax.broadcasted_iota(jnp.int32, (_BM, n), 1)
        nloc = jnp.min(jnp.where(dt == cmin, ciota, n), axis=1,
                       keepdims=True)
        idx2_ref[0, pl.ds(m0, _BM), :] = nloc
        d2sum = d2sum + jnp.sum(cmin)
        return qbest, qarg, d2sum

    init = (jnp.full((1, n), jnp.inf, f32), jnp.zeros((1, n), jnp.int32),
            jnp.float32(0.0))
    qbest, qarg, d2sum = lax.fori_loop(0, nt, tile1, init)

    idx1_ref[...] = qarg.reshape(1, 1, n)
    d1sum = jnp.sum(qbest)
    s1 = jnp.sum(xyz1, axis=0, keepdims=True)
    s2 = jnp.sum(xyz2_ref[0], axis=0, keepdims=True)

    out = jnp.concatenate([
        d1sum.reshape(1, 1), d2sum.reshape(1, 1),
        s1, s2, jnp.zeros((1, 8), f32),
    ], axis=1)
    out_ref[...] = out.reshape(1, 1, 16)


def _gather_body(tbl_ref, idx_ref, out_ref, tbl_v, idx_v, acc_v):
    f32 = jnp.float32
    w = lax.axis_index("s") * _NC + lax.axis_index("c")
    chunk = idx_v.shape[0]
    parts = _NW // tbl_ref.shape[0]
    job = w // parts
    pltpu.sync_copy(tbl_ref.at[job], tbl_v)
    pltpu.sync_copy(idx_ref.at[pl.ds(w * chunk, chunk)], idx_v)
    zero = jnp.zeros((16,), f32)

    def body(c, acc):
        a0, a1, a2 = acc
        iv = idx_v[pl.ds(c * 16, 16)] * 3
        a0 = a0 + plsc.load_gather(tbl_v, [iv])
        a1 = a1 + plsc.load_gather(tbl_v, [iv + 1])
        a2 = a2 + plsc.load_gather(tbl_v, [iv + 2])
        return (a0, a1, a2)

    a0, a1, a2 = lax.fori_loop(0, chunk // 16, body, (zero, zero, zero))
    acc_v[0, :] = a0
    acc_v[1, :] = a1
    acc_v[2, :] = a2
    pltpu.sync_copy(acc_v, out_ref.at[w])


@jax.jit
def kernel(xyz1, xyz2):
    b, n, _ = xyz1.shape
    m = xyz2.shape[1]
    f32 = jnp.float32
    xyz1 = xyz1.astype(f32)
    xyz2 = xyz2.astype(f32)
    pad1 = jnp.zeros((b, n, 5), f32)
    pad2 = jnp.zeros((b, m, 5), f32)
    x1b = jnp.concatenate([xyz1, pad1], axis=-1).astype(jnp.bfloat16)
    x2b = jnp.concatenate([xyz2, pad2], axis=-1).astype(jnp.bfloat16)
    sq1r = jnp.sum(xyz1 * xyz1, axis=-1)[:, None, :]

    partial, idx1, idx2 = pl.pallas_call(
        _chamfer_body,
        grid=(b,),
        in_specs=[
            pl.BlockSpec((1, n, 8), lambda i: (i, 0, 0)),
            pl.BlockSpec((1, m, 8), lambda i: (i, 0, 0)),
            pl.BlockSpec((1, 1, n), lambda i: (i, 0, 0)),
            pl.BlockSpec((1, n, 3), lambda i: (i, 0, 0)),
            pl.BlockSpec((1, m, 3), lambda i: (i, 0, 0)),
        ],
        out_specs=[
            pl.BlockSpec((1, 1, 16), lambda i: (i, 0, 0)),
            pl.BlockSpec((1, 1, n), lambda i: (i, 0, 0)),
            pl.BlockSpec((1, m, 1), lambda i: (i, 0, 0)),
        ],
        out_shape=[
            jax.ShapeDtypeStruct((b, 1, 16), f32),
            jax.ShapeDtypeStruct((b, 1, n), jnp.int32),
            jax.ShapeDtypeStruct((b, m, 1), jnp.int32),
        ],
    )(x1b, x2b, sq1r, xyz1, xyz2)

    jobs = 2 * b
    chunk = (jobs * n) // _NW
    tbl = jnp.stack([xyz2, xyz1], axis=1).reshape(jobs, n * 3)
    idxall = jnp.stack([idx1.reshape(b, n), idx2.reshape(b, m)],
                       axis=1).reshape(jobs * n)

    gath = pl.kernel(
        _gather_body,
        out_type=jax.ShapeDtypeStruct((_NW, 3, 16), f32),
        mesh=plsc.VectorSubcoreMesh(core_axis_name="c", subcore_axis_name="s"),
        compiler_params=pltpu.CompilerParams(needs_layout_passes=False),
        scratch_types=[
            pltpu.VMEM((n * 3,), f32),
            pltpu.VMEM((chunk,), jnp.int32),
            pltpu.VMEM((3, 16), f32),
        ],
    )(tbl, idxall)

    g = jnp.sum(gath.reshape(b, 2, _NW // jobs, 3, 16), axis=(0, 2, 4))
    g2 = g[0]
    g1 = g[1]

    tot = jnp.sum(partial[:, 0, :], axis=0)
    d1sum, d2sum = tot[0], tot[1]
    s1 = tot[2:5]
    s2 = tot[5:8]
    cd = d1sum / (b * n) + d2sum / (b * m)
    dev = (s1 - g2) / (b * n) + (s2 - g1) / (b * m)
    return jnp.concatenate([cd[None], dev], axis=0)


# speedup vs baseline: 1.9256x; 1.1022x over previous
import jax
import jax.numpy as jnp
from jax import lax
from jax.experimental import pallas as pl
from jax.experimental.pallas import tpu as pltpu
from jax.experimental.pallas import tpu_sc as plsc

_BM = 512

_NC = 2
_NS = 16
_NW = _NC * _NS


def _chamfer_body(xyz1_ref, xyz2_ref, sq1_ref, out_ref, idx1_ref, idx2_ref):
    f32 = jnp.float32
    xyz1 = xyz1_ref[0]
    sq1r = sq1_ref[0]
    n = xyz1.shape[0]
    m = xyz2_ref.shape[1]
    nt = m // _BM
    x1b = xyz1.astype(jnp.bfloat16)

    def tile1(t, carry):
        qbest, qarg, d2sum = carry
        m0 = t * _BM
        xyz2b = xyz2_ref[0, pl.ds(m0, _BM), :]
        inner = lax.dot_general(xyz2b.astype(jnp.bfloat16), x1b,
                                (((1,), (1,)), ((), ())),
                                preferred_element_type=f32)
        sq2c = jnp.sum(xyz2b * xyz2b, axis=1, keepdims=True)
        dt = (sq2c + sq1r) - 2.0 * inner
        qmin = jnp.min(dt, axis=0, keepdims=True)
        riota = lax.broadcasted_iota(jnp.int32, (_BM, n), 0)
        qloc = jnp.min(jnp.where(dt == qmin, riota, _BM), axis=0,
                       keepdims=True)
        cond = qmin < qbest
        qbest = jnp.where(cond, qmin, qbest)
        qarg = jnp.where(cond, qloc + m0, qarg)
        cmin = jnp.min(dt, axis=1, keepdims=True)
        ciota = lax.broadcasted_iota(jnp.int32, (_BM, n), 1)
        nloc = jnp.min(jnp.where(dt == cmin, ciota, n), axis=1,
                       keepdims=True)
        idx2_ref[0, pl.ds(m0, _BM), :] = nloc
        d2sum = d2sum + jnp.sum(cmin)
        return qbest, qarg, d2sum

    init = (jnp.full((1, n), jnp.inf, f32), jnp.zeros((1, n), jnp.int32),
            jnp.float32(0.0))
    qbest, qarg, d2sum = lax.fori_loop(0, nt, tile1, init)

    idx1_ref[...] = qarg.reshape(1, 1, n)
    d1sum = jnp.sum(qbest)
    s1 = jnp.sum(xyz1, axis=0, keepdims=True)
    s2 = jnp.sum(xyz2_ref[0], axis=0, keepdims=True)

    out = jnp.concatenate([
        d1sum.reshape(1, 1), d2sum.reshape(1, 1),
        s1, s2, jnp.zeros((1, 8), f32),
    ], axis=1)
    out_ref[...] = out.reshape(1, 1, 16)


def _gather_body(x1_ref, x2_ref, idx_ref, out_ref, tbl_v, idx_v, acc_v):
    f32 = jnp.float32
    w = lax.axis_index("s") * _NC + lax.axis_index("c")
    chunk = idx_v.shape[0]
    parts = _NW // (2 * x1_ref.shape[0])
    job = w // parts
    bb = job // 2
    side = job - 2 * bb

    @pl.when(side == 0)
    def _():
        pltpu.sync_copy(x2_ref.at[bb], tbl_v)

    @pl.when(side == 1)
    def _():
        pltpu.sync_copy(x1_ref.at[bb], tbl_v)

    pltpu.sync_copy(idx_ref.at[pl.ds(w * chunk, chunk)], idx_v)
    zero = jnp.zeros((16,), f32)

    def body(c, acc):
        a0, a1, a2 = acc
        iv = idx_v[pl.ds(c * 16, 16)] * 3
        a0 = a0 + plsc.load_gather(tbl_v, [iv])
        a1 = a1 + plsc.load_gather(tbl_v, [iv + 1])
        a2 = a2 + plsc.load_gather(tbl_v, [iv + 2])
        return (a0, a1, a2)

    a0, a1, a2 = lax.fori_loop(0, chunk // 16, body, (zero, zero, zero))
    acc_v[0, :] = a0
    acc_v[1, :] = a1
    acc_v[2, :] = a2
    pltpu.sync_copy(acc_v, out_ref.at[w])


@jax.jit
def kernel(xyz1, xyz2):
    b, n, _ = xyz1.shape
    m = xyz2.shape[1]
    f32 = jnp.float32
    xyz1 = xyz1.astype(f32)
    xyz2 = xyz2.astype(f32)
    sq1r = jnp.sum(xyz1 * xyz1, axis=-1)[:, None, :]

    partial, idx1, idx2 = pl.pallas_call(
        _chamfer_body,
        grid=(b,),
        in_specs=[
            pl.BlockSpec((1, n, 3), lambda i: (i, 0, 0)),
            pl.BlockSpec((1, m, 3), lambda i: (i, 0, 0)),
            pl.BlockSpec((1, 1, n), lambda i: (i, 0, 0)),
        ],
        out_specs=[
            pl.BlockSpec((1, 1, 16), lambda i: (i, 0, 0)),
            pl.BlockSpec((1, 1, n), lambda i: (i, 0, 0)),
            pl.BlockSpec((1, m, 1), lambda i: (i, 0, 0)),
        ],
        out_shape=[
            jax.ShapeDtypeStruct((b, 1, 16), f32),
            jax.ShapeDtypeStruct((b, 1, n), jnp.int32),
            jax.ShapeDtypeStruct((b, m, 1), jnp.int32),
        ],
    )(xyz1, xyz2, sq1r)

    chunk = (2 * b * n) // _NW
    idxall = jnp.stack([idx1.reshape(b, n), idx2.reshape(b, m)],
                       axis=1).reshape(2 * b * n)

    gath = pl.kernel(
        _gather_body,
        out_type=jax.ShapeDtypeStruct((_NW, 3, 16), f32),
        mesh=plsc.VectorSubcoreMesh(core_axis_name="c", subcore_axis_name="s"),
        compiler_params=pltpu.CompilerParams(needs_layout_passes=False),
        scratch_types=[
            pltpu.VMEM((n * 3,), f32),
            pltpu.VMEM((chunk,), jnp.int32),
            pltpu.VMEM((3, 16), f32),
        ],
    )(xyz1.reshape(b, n * 3), xyz2.reshape(b, m * 3), idxall)

    g = jnp.sum(gath.reshape(b, 2, _NW // (2 * b), 3, 16), axis=(0, 2, 4))
    g2 = g[0]
    g1 = g[1]

    tot = jnp.sum(partial[:, 0, :], axis=0)
    d1sum, d2sum = tot[0], tot[1]
    s1 = tot[2:5]
    s2 = tot[5:8]
    cd = d1sum / (b * n) + d2sum / (b * m)
    dev = (s1 - g2) / (b * n) + (s2 - g1) / (b * m)
    return jnp.concatenate([cd[None], dev], axis=0)


# speedup vs baseline: 1.9727x; 1.0245x over previous
import jax
import jax.numpy as jnp
from jax import lax
from jax.experimental import pallas as pl
from jax.experimental.pallas import tpu as pltpu
from jax.experimental.pallas import tpu_sc as plsc

_BM = 512

_NC = 2
_NS = 16
_NW = _NC * _NS
_L = 16


def _chamfer_body(xyz1_ref, xyz2_ref, sq1_ref, out_ref, idx1_ref, idx2_ref):
    f32 = jnp.float32
    xyz1 = xyz1_ref[0]
    sq1r = sq1_ref[0]
    n = xyz1.shape[0]
    m = xyz2_ref.shape[1]
    nt = m // _BM
    x1b = xyz1.astype(jnp.bfloat16)

    def tile1(t, carry):
        qbest, qarg, d2sum = carry
        m0 = t * _BM
        xyz2b = xyz2_ref[0, pl.ds(m0, _BM), :]
        inner = lax.dot_general(xyz2b.astype(jnp.bfloat16), x1b,
                                (((1,), (1,)), ((), ())),
                                preferred_element_type=f32)
        sq2c = jnp.sum(xyz2b * xyz2b, axis=1, keepdims=True)
        dt = (sq2c + sq1r) - 2.0 * inner
        qmin = jnp.min(dt, axis=0, keepdims=True)
        riota = lax.broadcasted_iota(jnp.int32, (_BM, n), 0)
        qloc = jnp.min(jnp.where(dt == qmin, riota, _BM), axis=0,
                       keepdims=True)
        cond = qmin < qbest
        qbest = jnp.where(cond, qmin, qbest)
        qarg = jnp.where(cond, qloc + m0, qarg)
        cmin = jnp.min(dt, axis=1, keepdims=True)
        ciota = lax.broadcasted_iota(jnp.int32, (_BM, n), 1)
        nloc = jnp.min(jnp.where(dt == cmin, ciota, n), axis=1,
                       keepdims=True)
        idx2_ref[0, pl.ds(m0, _BM), :] = nloc
        d2sum = d2sum + jnp.sum(cmin)
        return qbest, qarg, d2sum

    init = (jnp.full((1, n), jnp.inf, f32), jnp.zeros((1, n), jnp.int32),
            jnp.float32(0.0))
    qbest, qarg, d2sum = lax.fori_loop(0, nt, tile1, init)

    idx1_ref[...] = qarg.reshape(1, 1, n)
    d1sum = jnp.sum(qbest)
    s1 = jnp.sum(xyz1, axis=0, keepdims=True)
    s2 = jnp.sum(xyz2_ref[0], axis=0, keepdims=True)

    out = jnp.concatenate([
        d1sum.reshape(1, 1), d2sum.reshape(1, 1),
        s1, s2, jnp.zeros((1, 8), f32),
    ], axis=1)
    out_ref[...] = out.reshape(1, 1, 16)


def _hist_body(idx_ref, out_ref, idx_v, hist_v):
    f32 = jnp.float32
    w = lax.axis_index("s") * _NC + lax.axis_index("c")
    chunk = idx_v.shape[0]
    npts = hist_v.shape[0] // _L
    pltpu.sync_copy(idx_ref.at[pl.ds(w * chunk, chunk)], idx_v)

    zero16 = jnp.zeros((_L,), f32)

    def zbody(i, _):
        for k in range(8):
            hist_v[pl.ds(i * (8 * _L) + k * _L, _L)] = zero16
        return 0

    lax.fori_loop(0, hist_v.shape[0] // (8 * _L), zbody, 0)

    planes = lax.broadcasted_iota(jnp.int32, (_L,), 0) * npts
    ones16 = jnp.full((_L,), 1.0, f32)

    def sbody(c, _):
        for k in range(4):
            iv = idx_v[pl.ds(c * (4 * _L) + k * _L, _L)]
            plsc.addupdate_scatter(hist_v, [planes + iv], ones16)
        return 0

    lax.fori_loop(0, chunk // (4 * _L), sbody, 0)
    pltpu.sync_copy(hist_v, out_ref.at[w])


def _fold_body(cnt_ref, xyz1_ref, xyz2_ref, out_ref):
    f32 = jnp.float32
    n = xyz1_ref.shape[1]
    c1 = jnp.zeros((1, n), f32)
    c2 = jnp.zeros((1, n), f32)
    for plane in range(_L):
        blk = cnt_ref[:, pl.ds(plane * n, n)]
        c1 = c1 + jnp.sum(blk[0:4], axis=0, keepdims=True)
        c2 = c2 + jnp.sum(blk[4:8], axis=0, keepdims=True)
    n2sum = lax.dot_general(c1, xyz2_ref[0], (((1,), (0,)), ((), ())),
                            preferred_element_type=f32,
                            precision=lax.Precision.HIGHEST)
    n1sum = lax.dot_general(c2, xyz1_ref[0], (((1,), (0,)), ((), ())),
                            preferred_element_type=f32,
                            precision=lax.Precision.HIGHEST)
    out = jnp.concatenate([n1sum, n2sum, jnp.zeros((1, 2), f32)], axis=1)
    out_ref[...] = out.reshape(1, 1, 8)


@jax.jit
def kernel(xyz1, xyz2):
    b, n, _ = xyz1.shape
    m = xyz2.shape[1]
    f32 = jnp.float32
    xyz1 = xyz1.astype(f32)
    xyz2 = xyz2.astype(f32)
    sq1r = jnp.sum(xyz1 * xyz1, axis=-1)[:, None, :]

    partial, idx1, idx2 = pl.pallas_call(
        _chamfer_body,
        grid=(b,),
        in_specs=[
            pl.BlockSpec((1, n, 3), lambda i: (i, 0, 0)),
            pl.BlockSpec((1, m, 3), lambda i: (i, 0, 0)),
            pl.BlockSpec((1, 1, n), lambda i: (i, 0, 0)),
        ],
        out_specs=[
            pl.BlockSpec((1, 1, 16), lambda i: (i, 0, 0)),
            pl.BlockSpec((1, 1, n), lambda i: (i, 0, 0)),
            pl.BlockSpec((1, m, 1), lambda i: (i, 0, 0)),
        ],
        out_shape=[
            jax.ShapeDtypeStruct((b, 1, 16), f32),
            jax.ShapeDtypeStruct((b, 1, n), jnp.int32),
            jax.ShapeDtypeStruct((b, m, 1), jnp.int32),
        ],
    )(xyz1, xyz2, sq1r)

    chunk = (2 * b * n) // _NW
    idxall = jnp.stack([idx1.reshape(b, n), idx2.reshape(b, m)],
                       axis=1).reshape(2 * b * n)

    counts = pl.kernel(
        _hist_body,
        out_type=jax.ShapeDtypeStruct((_NW, _L * n), f32),
        mesh=plsc.VectorSubcoreMesh(core_axis_name="c", subcore_axis_name="s"),
        compiler_params=pltpu.CompilerParams(needs_layout_passes=False),
        scratch_types=[
            pltpu.VMEM((chunk,), jnp.int32),
            pltpu.VMEM((_L * n,), f32),
        ],
    )(idxall)

    partial2 = pl.pallas_call(
        _fold_body,
        grid=(b,),
        in_specs=[
            pl.BlockSpec((8, _L * n), lambda i: (i, 0)),
            pl.BlockSpec((1, n, 3), lambda i: (i, 0, 0)),
            pl.BlockSpec((1, m, 3), lambda i: (i, 0, 0)),
        ],
        out_specs=pl.BlockSpec((1, 1, 8), lambda i: (i, 0, 0)),
        out_shape=jax.ShapeDtypeStruct((b, 1, 8), f32),
    )(counts, xyz1, xyz2)

    tot2 = jnp.sum(partial2[:, 0, :], axis=0)
    g1 = tot2[0:3]
    g2 = tot2[3:6]

    tot = jnp.sum(partial[:, 0, :], axis=0)
    d1sum, d2sum = tot[0], tot[1]
    s1 = tot[2:5]
    s2 = tot[5:8]
    cd = d1sum / (b * n) + d2sum / (b * m)
    dev = (s1 - g2) / (b * n) + (s2 - g1) / (b * m)
    return jnp.concatenate([cd[None], dev], axis=0)


# speedup vs baseline: 2.0163x; 1.0221x over previous
import jax
import jax.numpy as jnp
from jax import lax
from jax.experimental import pallas as pl
from jax.experimental.pallas import tpu as pltpu
from jax.experimental.pallas import tpu_sc as plsc

_BM = 512

_NC = 2
_NS = 16
_NW = _NC * _NS
_L = 16


def _chamfer_body(xyz1_ref, xyz2_ref, sq1_ref, out_ref, idx1_ref, idx2_ref,
                  idx2col_ref):
    f32 = jnp.float32
    xyz1 = xyz1_ref[0]
    sq1r = sq1_ref[0]
    n = xyz1.shape[0]
    m = xyz2_ref.shape[1]
    nt = m // _BM
    x1b = xyz1.astype(jnp.bfloat16)

    def tile1(t, carry):
        qbest, qarg, d2sum = carry
        m0 = t * _BM
        xyz2b = xyz2_ref[0, pl.ds(m0, _BM), :]
        inner = lax.dot_general(xyz2b.astype(jnp.bfloat16), x1b,
                                (((1,), (1,)), ((), ())),
                                preferred_element_type=f32)
        sq2c = jnp.sum(xyz2b * xyz2b, axis=1, keepdims=True)
        dt = (sq2c + sq1r) - 2.0 * inner
        qmin = jnp.min(dt, axis=0, keepdims=True)
        riota = lax.broadcasted_iota(jnp.int32, (_BM, n), 0)
        qloc = jnp.min(jnp.where(dt == qmin, riota, _BM), axis=0,
                       keepdims=True)
        cond = qmin < qbest
        qbest = jnp.where(cond, qmin, qbest)
        qarg = jnp.where(cond, qloc + m0, qarg)
        cmin = jnp.min(dt, axis=1, keepdims=True)
        ciota = lax.broadcasted_iota(jnp.int32, (_BM, n), 1)
        nloc = jnp.min(jnp.where(dt == cmin, ciota, n), axis=1,
                       keepdims=True)
        idx2col_ref[pl.ds(m0, _BM), :] = nloc
        d2sum = d2sum + jnp.sum(cmin)
        return qbest, qarg, d2sum

    init = (jnp.full((1, n), jnp.inf, f32), jnp.zeros((1, n), jnp.int32),
            jnp.float32(0.0))
    qbest, qarg, d2sum = lax.fori_loop(0, nt, tile1, init)

    idx1_ref[...] = qarg.reshape(1, 1, n)
    idx2_ref[...] = lax.transpose(idx2col_ref[...], (1, 0)).reshape(1, 1, m)
    d1sum = jnp.sum(qbest)
    s1 = jnp.sum(xyz1, axis=0, keepdims=True)
    s2 = jnp.sum(xyz2_ref[0], axis=0, keepdims=True)

    out = jnp.concatenate([
        d1sum.reshape(1, 1), d2sum.reshape(1, 1),
        s1, s2, jnp.zeros((1, 8), f32),
    ], axis=1)
    out_ref[...] = out.reshape(1, 1, 16)


def _hist_body(idx_ref, out_ref, idx_v, hist_v):
    f32 = jnp.float32
    w = lax.axis_index("s") * _NC + lax.axis_index("c")
    chunk = idx_v.shape[0]
    npts = hist_v.shape[0] // _L
    pltpu.sync_copy(idx_ref.at[pl.ds(w * chunk, chunk)], idx_v)

    zero16 = jnp.zeros((_L,), f32)

    def zbody(i, _):
        for k in range(8):
            hist_v[pl.ds(i * (8 * _L) + k * _L, _L)] = zero16
        return 0

    lax.fori_loop(0, hist_v.shape[0] // (8 * _L), zbody, 0)

    planes = lax.broadcasted_iota(jnp.int32, (_L,), 0) * npts
    ones16 = jnp.full((_L,), 1.0, f32)

    def sbody(c, _):
        for k in range(4):
            iv = idx_v[pl.ds(c * (4 * _L) + k * _L, _L)]
            plsc.addupdate_scatter(hist_v, [planes + iv], ones16)
        return 0

    lax.fori_loop(0, chunk // (4 * _L), sbody, 0)
    pltpu.sync_copy(hist_v, out_ref.at[w])


def _fold_body(cnt_ref, xyz1_ref, xyz2_ref, out_ref):
    f32 = jnp.float32
    n = xyz1_ref.shape[1]
    c1 = jnp.zeros((1, n), f32)
    c2 = jnp.zeros((1, n), f32)
    for plane in range(_L):
        blk = cnt_ref[:, pl.ds(plane * n, n)]
        c1 = c1 + jnp.sum(blk[0:4], axis=0, keepdims=True)
        c2 = c2 + jnp.sum(blk[4:8], axis=0, keepdims=True)
    n2sum = lax.dot_general(c1, xyz2_ref[0], (((1,), (0,)), ((), ())),
                            preferred_element_type=f32,
                            precision=lax.Precision.HIGHEST)
    n1sum = lax.dot_general(c2, xyz1_ref[0], (((1,), (0,)), ((), ())),
                            preferred_element_type=f32,
                            precision=lax.Precision.HIGHEST)
    out = jnp.concatenate([n1sum, n2sum, jnp.zeros((1, 2), f32)], axis=1)
    out_ref[...] = out.reshape(1, 1, 8)


@jax.jit
def kernel(xyz1, xyz2):
    b, n, _ = xyz1.shape
    m = xyz2.shape[1]
    f32 = jnp.float32
    xyz1 = xyz1.astype(f32)
    xyz2 = xyz2.astype(f32)
    sq1r = jnp.sum(xyz1 * xyz1, axis=-1)[:, None, :]

    partial, idx1, idx2 = pl.pallas_call(
        _chamfer_body,
        grid=(b,),
        in_specs=[
            pl.BlockSpec((1, n, 3), lambda i: (i, 0, 0)),
            pl.BlockSpec((1, m, 3), lambda i: (i, 0, 0)),
            pl.BlockSpec((1, 1, n), lambda i: (i, 0, 0)),
        ],
        out_specs=[
            pl.BlockSpec((1, 1, 16), lambda i: (i, 0, 0)),
            pl.BlockSpec((1, 1, n), lambda i: (i, 0, 0)),
            pl.BlockSpec((1, 1, m), lambda i: (i, 0, 0)),
        ],
        out_shape=[
            jax.ShapeDtypeStruct((b, 1, 16), f32),
            jax.ShapeDtypeStruct((b, 1, n), jnp.int32),
            jax.ShapeDtypeStruct((b, 1, m), jnp.int32),
        ],
        scratch_shapes=[pltpu.VMEM((m, 1), jnp.int32)],
    )(xyz1, xyz2, sq1r)

    chunk = (2 * b * n) // _NW
    idxall = jnp.concatenate([idx1, idx2], axis=1).reshape(2 * b * n)

    counts = pl.kernel(
        _hist_body,
        out_type=jax.ShapeDtypeStruct((_NW, _L * n), f32),
        mesh=plsc.VectorSubcoreMesh(core_axis_name="c", subcore_axis_name="s"),
        compiler_params=pltpu.CompilerParams(needs_layout_passes=False),
        scratch_types=[
            pltpu.VMEM((chunk,), jnp.int32),
            pltpu.VMEM((_L * n,), f32),
        ],
    )(idxall)

    partial2 = pl.pallas_call(
        _fold_body,
        grid=(b,),
        in_specs=[
            pl.BlockSpec((8, _L * n), lambda i: (i, 0)),
            pl.BlockSpec((1, n, 3), lambda i: (i, 0, 0)),
            pl.BlockSpec((1, m, 3), lambda i: (i, 0, 0)),
        ],
        out_specs=pl.BlockSpec((1, 1, 8), lambda i: (i, 0, 0)),
        out_shape=jax.ShapeDtypeStruct((b, 1, 8), f32),
    )(counts, xyz1, xyz2)

    tot2 = jnp.sum(partial2[:, 0, :], axis=0)
    g1 = tot2[0:3]
    g2 = tot2[3:6]

    tot = jnp.sum(partial[:, 0, :], axis=0)
    d1sum, d2sum = tot[0], tot[1]
    s1 = tot[2:5]
    s2 = tot[5:8]
    cd = d1sum / (b * n) + d2sum / (b * m)
    dev = (s1 - g2) / (b * n) + (s2 - g1) / (b * m)
    return jnp.concatenate([cd[None], dev], axis=0)


# speedup vs baseline: 2.1375x; 1.0601x over previous
import jax
import jax.numpy as jnp
from jax import lax
from jax.experimental import pallas as pl
from jax.experimental.pallas import tpu as pltpu
from jax.experimental.pallas import tpu_sc as plsc

_BM = 1024

_NC = 2
_NS = 16
_NW = _NC * _NS
_L = 16


def _chamfer_body(xyz1_ref, xyz2_ref, sq1_ref, out_ref, idx1_ref, idx2_ref,
                  idx2col_ref):
    f32 = jnp.float32
    xyz1 = xyz1_ref[0]
    sq1r = sq1_ref[0]
    n = xyz1.shape[0]
    m = xyz2_ref.shape[1]
    nt = m // _BM
    x1b = xyz1.astype(jnp.bfloat16)

    def tile1(t, carry):
        qbest, qarg, d2sum = carry
        m0 = t * _BM
        xyz2b = xyz2_ref[0, pl.ds(m0, _BM), :]
        inner = lax.dot_general(xyz2b.astype(jnp.bfloat16), x1b,
                                (((1,), (1,)), ((), ())),
                                preferred_element_type=f32)
        sq2c = jnp.sum(xyz2b * xyz2b, axis=1, keepdims=True)
        dt = (sq2c + sq1r) - 2.0 * inner
        qmin = jnp.min(dt, axis=0, keepdims=True)
        riota = lax.broadcasted_iota(jnp.int32, (_BM, n), 0)
        qloc = jnp.min(jnp.where(dt == qmin, riota, _BM), axis=0,
                       keepdims=True)
        cond = qmin < qbest
        qbest = jnp.where(cond, qmin, qbest)
        qarg = jnp.where(cond, qloc + m0, qarg)
        cmin = jnp.min(dt, axis=1, keepdims=True)
        ciota = lax.broadcasted_iota(jnp.int32, (_BM, n), 1)
        nloc = jnp.min(jnp.where(dt == cmin, ciota, n), axis=1,
                       keepdims=True)
        idx2col_ref[pl.ds(m0, _BM), :] = nloc
        d2sum = d2sum + jnp.sum(cmin)
        return qbest, qarg, d2sum

    init = (jnp.full((1, n), jnp.inf, f32), jnp.zeros((1, n), jnp.int32),
            jnp.float32(0.0))
    qbest, qarg, d2sum = lax.fori_loop(0, nt, tile1, init)

    idx1_ref[...] = qarg.reshape(1, 1, n)
    idx2_ref[...] = lax.transpose(idx2col_ref[...], (1, 0)).reshape(1, 1, m)
    d1sum = jnp.sum(qbest)
    s1 = jnp.sum(xyz1, axis=0, keepdims=True)
    s2 = jnp.sum(xyz2_ref[0], axis=0, keepdims=True)

    out = jnp.concatenate([
        d1sum.reshape(1, 1), d2sum.reshape(1, 1),
        s1, s2, jnp.zeros((1, 8), f32),
    ], axis=1)
    out_ref[...] = out.reshape(1, 1, 16)


def _hist_body(idx_ref, out_ref, idx_v, hist_v):
    f32 = jnp.float32
    w = lax.axis_index("s") * _NC + lax.axis_index("c")
    chunk = idx_v.shape[0]
    npts = hist_v.shape[0] // _L
    pltpu.sync_copy(idx_ref.at[pl.ds(w * chunk, chunk)], idx_v)

    zero16 = jnp.zeros((_L,), f32)

    def zbody(i, _):
        for k in range(8):
            hist_v[pl.ds(i * (8 * _L) + k * _L, _L)] = zero16
        return 0

    lax.fori_loop(0, hist_v.shape[0] // (8 * _L), zbody, 0)

    planes = lax.broadcasted_iota(jnp.int32, (_L,), 0) * npts
    ones16 = jnp.full((_L,), 1.0, f32)

    def sbody(c, _):
        for k in range(4):
            iv = idx_v[pl.ds(c * (4 * _L) + k * _L, _L)]
            plsc.addupdate_scatter(hist_v, [planes + iv], ones16)
        return 0

    lax.fori_loop(0, chunk // (4 * _L), sbody, 0)
    pltpu.sync_copy(hist_v, out_ref.at[w])


def _fold_body(cnt_ref, xyz1_ref, xyz2_ref, out_ref):
    f32 = jnp.float32
    n = xyz1_ref.shape[1]
    c1 = jnp.zeros((1, n), f32)
    c2 = jnp.zeros((1, n), f32)
    for plane in range(_L):
        blk = cnt_ref[:, pl.ds(plane * n, n)]
        c1 = c1 + jnp.sum(blk[0:4], axis=0, keepdims=True)
        c2 = c2 + jnp.sum(blk[4:8], axis=0, keepdims=True)
    n2sum = lax.dot_general(c1, xyz2_ref[0], (((1,), (0,)), ((), ())),
                            preferred_element_type=f32,
                            precision=lax.Precision.HIGHEST)
    n1sum = lax.dot_general(c2, xyz1_ref[0], (((1,), (0,)), ((), ())),
                            preferred_element_type=f32,
                            precision=lax.Precision.HIGHEST)
    out = jnp.concatenate([n1sum, n2sum, jnp.zeros((1, 2), f32)], axis=1)
    out_ref[...] = out.reshape(1, 1, 8)


@jax.jit
def kernel(xyz1, xyz2):
    b, n, _ = xyz1.shape
    m = xyz2.shape[1]
    f32 = jnp.float32
    xyz1 = xyz1.astype(f32)
    xyz2 = xyz2.astype(f32)
    sq1r = jnp.sum(xyz1 * xyz1, axis=-1)[:, None, :]

    partial, idx1, idx2 = pl.pallas_call(
        _chamfer_body,
        grid=(b,),
        in_specs=[
            pl.BlockSpec((1, n, 3), lambda i: (i, 0, 0)),
            pl.BlockSpec((1, m, 3), lambda i: (i, 0, 0)),
            pl.BlockSpec((1, 1, n), lambda i: (i, 0, 0)),
        ],
        out_specs=[
            pl.BlockSpec((1, 1, 16), lambda i: (i, 0, 0)),
            pl.BlockSpec((1, 1, n), lambda i: (i, 0, 0)),
            pl.BlockSpec((1, 1, m), lambda i: (i, 0, 0)),
        ],
        out_shape=[
            jax.ShapeDtypeStruct((b, 1, 16), f32),
            jax.ShapeDtypeStruct((b, 1, n), jnp.int32),
            jax.ShapeDtypeStruct((b, 1, m), jnp.int32),
        ],
        scratch_shapes=[pltpu.VMEM((m, 1), jnp.int32)],
    )(xyz1, xyz2, sq1r)

    chunk = (2 * b * n) // _NW
    idxall = jnp.concatenate([idx1, idx2], axis=1).reshape(2 * b * n)

    counts = pl.kernel(
        _hist_body,
        out_type=jax.ShapeDtypeStruct((_NW, _L * n), f32),
        mesh=plsc.VectorSubcoreMesh(core_axis_name="c", subcore_axis_name="s"),
        compiler_params=pltpu.CompilerParams(needs_layout_passes=False),
        scratch_types=[
            pltpu.VMEM((chunk,), jnp.int32),
            pltpu.VMEM((_L * n,), f32),
        ],
    )(idxall)

    partial2 = pl.pallas_call(
        _fold_body,
        grid=(b,),
        in_specs=[
            pl.BlockSpec((8, _L * n), lambda i: (i, 0)),
            pl.BlockSpec((1, n, 3), lambda i: (i, 0, 0)),
            pl.BlockSpec((1, m, 3), lambda i: (i, 0, 0)),
        ],
        out_specs=pl.BlockSpec((1, 1, 8), lambda i: (i, 0, 0)),
        out_shape=jax.ShapeDtypeStruct((b, 1, 8), f32),
    )(counts, xyz1, xyz2)

    tot2 = jnp.sum(partial2[:, 0, :], axis=0)
    g1 = tot2[0:3]
    g2 = tot2[3:6]

    tot = jnp.sum(partial[:, 0, :], axis=0)
    d1sum, d2sum = tot[0], tot[1]
    s1 = tot[2:5]
    s2 = tot[5:8]
    cd = d1sum / (b * n) + d2sum / (b * m)
    dev = (s1 - g2) / (b * n) + (s2 - g1) / (b * m)
    return jnp.concatenate([cd[None], dev], axis=0)


# speedup vs baseline: 2.1797x; 1.0197x over previous
import jax
import jax.numpy as jnp
from jax import lax
from jax.experimental import pallas as pl
from jax.experimental.pallas import tpu as pltpu
from jax.experimental.pallas import tpu_sc as plsc

_BM = 2048

_NC = 2
_NS = 16
_NW = _NC * _NS
_L = 16


def _chamfer_body(xyz1_ref, xyz2_ref, sq1_ref, out_ref, idx1_ref, idx2_ref,
                  idx2col_ref):
    f32 = jnp.float32
    xyz1 = xyz1_ref[0]
    sq1r = sq1_ref[0]
    n = xyz1.shape[0]
    m = xyz2_ref.shape[1]
    nt = m // _BM
    x1b = xyz1.astype(jnp.bfloat16)

    def tile1(t, carry):
        qbest, qarg, d2sum = carry
        m0 = t * _BM
        xyz2b = xyz2_ref[0, pl.ds(m0, _BM), :]
        inner = lax.dot_general(xyz2b.astype(jnp.bfloat16), x1b,
                                (((1,), (1,)), ((), ())),
                                preferred_element_type=f32)
        sq2c = jnp.sum(xyz2b * xyz2b, axis=1, keepdims=True)
        dt = (sq2c + sq1r) - 2.0 * inner
        qmin = jnp.min(dt, axis=0, keepdims=True)
        riota = lax.broadcasted_iota(jnp.int32, (_BM, n), 0)
        qloc = jnp.min(jnp.where(dt == qmin, riota, _BM), axis=0,
                       keepdims=True)
        cond = qmin < qbest
        qbest = jnp.where(cond, qmin, qbest)
        qarg = jnp.where(cond, qloc + m0, qarg)
        cmin = jnp.min(dt, axis=1, keepdims=True)
        ciota = lax.broadcasted_iota(jnp.int32, (_BM, n), 1)
        nloc = jnp.min(jnp.where(dt == cmin, ciota, n), axis=1,
                       keepdims=True)
        idx2col_ref[pl.ds(m0, _BM), :] = nloc
        d2sum = d2sum + jnp.sum(cmin)
        return qbest, qarg, d2sum

    init = (jnp.full((1, n), jnp.inf, f32), jnp.zeros((1, n), jnp.int32),
            jnp.float32(0.0))
    qbest, qarg, d2sum = lax.fori_loop(0, nt, tile1, init)

    idx1_ref[...] = qarg.reshape(1, 1, n)
    idx2_ref[...] = lax.transpose(idx2col_ref[...], (1, 0)).reshape(1, 1, m)
    d1sum = jnp.sum(qbest)
    s1 = jnp.sum(xyz1, axis=0, keepdims=True)
    s2 = jnp.sum(xyz2_ref[0], axis=0, keepdims=True)

    out = jnp.concatenate([
        d1sum.reshape(1, 1), d2sum.reshape(1, 1),
        s1, s2, jnp.zeros((1, 8), f32),
    ], axis=1)
    out_ref[...] = out.reshape(1, 1, 16)


def _hist_body(idx_ref, out_ref, idx_v, hist_v):
    f32 = jnp.float32
    w = lax.axis_index("s") * _NC + lax.axis_index("c")
    chunk = idx_v.shape[0]
    npts = hist_v.shape[0] // _L
    pltpu.sync_copy(idx_ref.at[pl.ds(w * chunk, chunk)], idx_v)

    zero16 = jnp.zeros((_L,), f32)

    def zbody(i, _):
        for k in range(8):
            hist_v[pl.ds(i * (8 * _L) + k * _L, _L)] = zero16
        return 0

    lax.fori_loop(0, hist_v.shape[0] // (8 * _L), zbody, 0)

    planes = lax.broadcasted_iota(jnp.int32, (_L,), 0) * npts
    ones16 = jnp.full((_L,), 1.0, f32)

    def sbody(c, _):
        for k in range(4):
            iv = idx_v[pl.ds(c * (4 * _L) + k * _L, _L)]
            plsc.addupdate_scatter(hist_v, [planes + iv], ones16)
        return 0

    lax.fori_loop(0, chunk // (4 * _L), sbody, 0)
    pltpu.sync_copy(hist_v, out_ref.at[w])


def _fold_body(cnt_ref, xyz1_ref, xyz2_ref, out_ref):
    f32 = jnp.float32
    n = xyz1_ref.shape[1]
    c1 = jnp.zeros((1, n), f32)
    c2 = jnp.zeros((1, n), f32)
    for plane in range(_L):
        blk = cnt_ref[:, pl.ds(plane * n, n)]
        c1 = c1 + jnp.sum(blk[0:4], axis=0, keepdims=True)
        c2 = c2 + jnp.sum(blk[4:8], axis=0, keepdims=True)
    n2sum = lax.dot_general(c1, xyz2_ref[0], (((1,), (0,)), ((), ())),
                            preferred_element_type=f32,
                            precision=lax.Precision.HIGHEST)
    n1sum = lax.dot_general(c2, xyz1_ref[0], (((1,), (0,)), ((), ())),
                            preferred_element_type=f32,
                            precision=lax.Precision.HIGHEST)
    out = jnp.concatenate([n1sum, n2sum, jnp.zeros((1, 2), f32)], axis=1)
    out_ref[...] = out.reshape(1, 1, 8)


@jax.jit
def kernel(xyz1, xyz2):
    b, n, _ = xyz1.shape
    m = xyz2.shape[1]
    f32 = jnp.float32
    xyz1 = xyz1.astype(f32)
    xyz2 = xyz2.astype(f32)
    sq1r = jnp.sum(xyz1 * xyz1, axis=-1)[:, None, :]

    partial, idx1, idx2 = pl.pallas_call(
        _chamfer_body,
        grid=(b,),
        in_specs=[
            pl.BlockSpec((1, n, 3), lambda i: (i, 0, 0)),
            pl.BlockSpec((1, m, 3), lambda i: (i, 0, 0)),
            pl.BlockSpec((1, 1, n), lambda i: (i, 0, 0)),
        ],
        out_specs=[
            pl.BlockSpec((1, 1, 16), lambda i: (i, 0, 0)),
            pl.BlockSpec((1, 1, n), lambda i: (i, 0, 0)),
            pl.BlockSpec((1, 1, m), lambda i: (i, 0, 0)),
        ],
        out_shape=[
            jax.ShapeDtypeStruct((b, 1, 16), f32),
            jax.ShapeDtypeStruct((b, 1, n), jnp.int32),
            jax.ShapeDtypeStruct((b, 1, m), jnp.int32),
        ],
        scratch_shapes=[pltpu.VMEM((m, 1), jnp.int32)],
    )(xyz1, xyz2, sq1r)

    chunk = (2 * b * n) // _NW
    idxall = jnp.concatenate([idx1, idx2], axis=1).reshape(2 * b * n)

    counts = pl.kernel(
        _hist_body,
        out_type=jax.ShapeDtypeStruct((_NW, _L * n), f32),
        mesh=plsc.VectorSubcoreMesh(core_axis_name="c", subcore_axis_name="s"),
        compiler_params=pltpu.CompilerParams(needs_layout_passes=False),
        scratch_types=[
            pltpu.VMEM((chunk,), jnp.int32),
            pltpu.VMEM((_L * n,), f32),
        ],
    )(idxall)

    partial2 = pl.pallas_call(
        _fold_body,
        grid=(b,),
        in_specs=[
            pl.BlockSpec((8, _L * n), lambda i: (i, 0)),
            pl.BlockSpec((1, n, 3), lambda i: (i, 0, 0)),
            pl.BlockSpec((1, m, 3), lambda i: (i, 0, 0)),
        ],
        out_specs=pl.BlockSpec((1, 1, 8), lambda i: (i, 0, 0)),
        out_shape=jax.ShapeDtypeStruct((b, 1, 8), f32),
    )(counts, xyz1, xyz2)

    tot2 = jnp.sum(partial2[:, 0, :], axis=0)
    g1 = tot2[0:3]
    g2 = tot2[3:6]

    tot = jnp.sum(partial[:, 0, :], axis=0)
    d1sum, d2sum = tot[0], tot[1]
    s1 = tot[2:5]
    s2 = tot[5:8]
    cd = d1sum / (b * n) + d2sum / (b * m)
    dev = (s1 - g2) / (b * n) + (s2 - g1) / (b * m)
    return jnp.concatenate([cd[None], dev], axis=0)
